# baseline (speedup 1.0000x reference)
import jax
import jax.numpy as jnp
from jax.experimental import pallas as pl


def _pe_add_kernel(x_ref, pe_ref, o_ref):
    i = pl.program_id(0)
    BS = x_ref.shape[0]
    pe = pe_ref[pl.ds(i * BS, BS), :]
    for b in range(x_ref.shape[1]):
        o_ref[:, b, :] = x_ref[:, b, :] + pe


def kernel(x, pos_emb):
    S, B, D = x.shape
    BS = 128
    return pl.pallas_call(
        _pe_add_kernel,
        grid=(S // BS,),
        in_specs=[
            pl.BlockSpec((BS, B, D), lambda i: (i, 0, 0)),
            pl.BlockSpec((S, D), lambda i: (0, 0)),
        ],
        out_specs=pl.BlockSpec((BS, B, D), lambda i: (i, 0, 0)),
        out_shape=jax.ShapeDtypeStruct((S, B, D), x.dtype),
    )(x, pos_emb[:S])


# speedup vs baseline: 1.0144x; 1.0144x over previous
import jax
import jax.numpy as jnp
from jax.experimental import pallas as pl


def _pe_add_kernel(x_ref, pe_ref, o_ref):
    pe = pe_ref[...]
    for b in range(x_ref.shape[1]):
        o_ref[:, b, :] = x_ref[:, b, :] + pe


def kernel(x, pos_emb):
    S, B, D = x.shape
    BS = 256
    return pl.pallas_call(
        _pe_add_kernel,
        grid=(S // BS,),
        in_specs=[
            pl.BlockSpec((BS, B, D), lambda i: (i, 0, 0)),
            pl.BlockSpec((BS, D), lambda i: (i, 0)),
        ],
        out_specs=pl.BlockSpec((BS, B, D), lambda i: (i, 0, 0)),
        out_shape=jax.ShapeDtypeStruct((S, B, D), x.dtype),
    )(x, pos_emb[:S])
